# baseline (speedup 1.0000x reference)
import jax
import jax.numpy as jnp
from jax import lax
from jax.experimental import pallas as pl
from jax.experimental.pallas import tpu as pltpu
from jax.experimental.pallas import tpu_sc as plsc

MAX_LEN = 8192
EMB_DIM = 64
PAD_DIM = 128
N_ROWS = 4096
N_COLS = 200
HALF = N_COLS // 2

NC = 2
NS = 16
NW = NC * NS
ROWS_PER_W = N_ROWS // NW


def _body(idx_hbm, table_hbm, out_hbm,
          i0, i1, g0, g1, b0, b1,
          si0, si1, sg0, sg1, sw0, sw1):
    wid = lax.axis_index("s") * NC + lax.axis_index("c")
    base = wid * ROWS_PER_W

    idx_bufs = (i0, i1)
    g_bufs = (g0, g1)
    b_bufs = (b0, b1)
    i_sems = (si0, si1)
    g_sems = (sg0, sg1)
    w_sems = (sw0, sw1)

    def idx_load(r, p):
        pltpu.async_copy(idx_hbm.at[wid, pl.ds(2 * r, 2)], idx_bufs[p], i_sems[p])

    def idx_wait(p):
        pltpu.make_async_copy(idx_hbm.at[0, pl.ds(0, 2)], idx_bufs[p], i_sems[p]).wait()

    def gather(p):
        pltpu.async_copy(table_hbm.at[idx_bufs[p].at[0]],
                         g_bufs[p].at[pl.ds(0, HALF)], g_sems[p])
        pltpu.async_copy(table_hbm.at[idx_bufs[p].at[1]],
                         g_bufs[p].at[pl.ds(HALF, HALF)], g_sems[p])

    def gather_drain(p):
        pltpu.make_async_copy(table_hbm.at[idx_bufs[p].at[0]],
                              g_bufs[p].at[pl.ds(0, HALF)], g_sems[p]).wait()
        pltpu.make_async_copy(table_hbm.at[idx_bufs[p].at[0]],
                              g_bufs[p].at[pl.ds(HALF, HALF)], g_sems[p]).wait()

    def compact(p):
        g, b = g_bufs[p], b_bufs[p]

        def rows4(k, carry):
            i = k * 4
            for di in range(4):
                for c in range(4):
                    b[i + di, pl.ds(c * 16, 16)] = g[i + di, pl.ds(c * 16, 16)]
            return carry

        lax.fori_loop(0, N_COLS // 4, rows4, 0)

    def write(r, p):
        pltpu.async_copy(b_bufs[p], out_hbm.at[base + r], w_sems[p])

    def write_wait(p):
        pltpu.make_async_copy(b_bufs[p], out_hbm.at[0], w_sems[p]).wait()

    idx_load(0, 0)
    idx_load(1, 1)
    idx_wait(0)
    gather(0)
    idx_wait(1)
    gather(1)

    def step(rr, carry):
        for p in range(2):
            r = rr * 2 + p
            gather_drain(p)

            @pl.when(r >= 2)
            def _():
                write_wait(p)

            compact(p)
            write(r, p)

            @pl.when(r + 2 < ROWS_PER_W)
            def _():
                idx_load(r + 2, p)
                idx_wait(p)
                gather(p)

        return carry

    lax.fori_loop(0, ROWS_PER_W // 2, step, 0)
    write_wait(0)
    write_wait(1)


@jax.jit
def _gather_op(positions, table_padded):
    mesh = plsc.VectorSubcoreMesh(core_axis_name="c", subcore_axis_name="s")
    idx = positions.reshape(NW, ROWS_PER_W * 2, HALF)
    out = pl.kernel(
        _body,
        out_type=jax.ShapeDtypeStruct((N_ROWS, N_COLS, EMB_DIM), jnp.float32),
        mesh=mesh,
        scratch_types=[
            pltpu.VMEM((2, HALF), jnp.int32),
            pltpu.VMEM((2, HALF), jnp.int32),
            pltpu.VMEM((N_COLS, PAD_DIM), jnp.float32),
            pltpu.VMEM((N_COLS, PAD_DIM), jnp.float32),
            pltpu.VMEM((N_COLS, EMB_DIM), jnp.float32),
            pltpu.VMEM((N_COLS, EMB_DIM), jnp.float32),
            pltpu.SemaphoreType.DMA,
            pltpu.SemaphoreType.DMA,
            pltpu.SemaphoreType.DMA,
            pltpu.SemaphoreType.DMA,
            pltpu.SemaphoreType.DMA,
            pltpu.SemaphoreType.DMA,
        ],
        compiler_params=pltpu.CompilerParams(use_tc_tiling_on_sc=True),
    )(idx, table_padded)
    return out


def kernel(positions, table):
    table_padded = jnp.pad(table, ((0, 0), (0, PAD_DIM - EMB_DIM)))
    return _gather_op(positions, table_padded)
